# baseline (speedup 1.0000x reference)
import functools
import jax
import jax.numpy as jnp
from jax import lax
from jax.experimental import pallas as pl
from jax.experimental.pallas import tpu as pltpu
from jax.experimental.pallas import tpu_sc as plsc

D = 64
SCALE = 8.0
NC, NS, L = 2, 16, 16
NW = NC * NS
NB = 200
BLK = 128
NBUF = 4
LOOKAHEAD = 2

_mesh = plsc.VectorSubcoreMesh(core_axis_name="c", subcore_axis_name="s")


@functools.partial(
    pl.kernel,
    mesh=_mesh,
    out_type=jax.ShapeDtypeStruct((4096, NB, D), jnp.float32),
    scratch_types=[
        [pltpu.VMEM((BLK,), jnp.int32) for _ in range(NBUF)],
        [pltpu.VMEM((BLK, D), jnp.float32) for _ in range(NBUF)],
        [pltpu.SemaphoreType.DMA for _ in range(NBUF)],
        [pltpu.SemaphoreType.DMA for _ in range(NBUF)],
        [pltpu.SemaphoreType.DMA for _ in range(NBUF)],
    ],
    compiler_params=pltpu.CompilerParams(use_tc_tiling_on_sc=False),
)
def _emb_lookup(tok_hbm, table_hbm, out_hbm, ibuf, rows, isem, gsem, wsem):
    wid = lax.axis_index("s") * NC + lax.axis_index("c")

    def idx_load(g, b, use_sem):
        ts = lax.div(g, 8)
        si = lax.rem(g, 8)
        if use_sem:
            pltpu.async_copy(tok_hbm.at[ts, wid, si], ibuf[b], isem[b])
        else:
            pltpu.sync_copy(tok_hbm.at[ts, wid, si], ibuf[b])

    def gather_wait(b):
        pltpu.make_async_copy(table_hbm.at[pl.ds(0, BLK)], rows[b], gsem[b]).wait()

    def write_wait(b):
        pltpu.make_async_copy(table_hbm.at[pl.ds(0, BLK)], rows[b], wsem[b]).wait()

    def idx_wait(b):
        pltpu.make_async_copy(tok_hbm.at[0, 0, 0], ibuf[b], isem[b]).wait()

    idx_load(0, 0, False)
    idx_load(1, 1, False)
    idx_load(2, 2, True)
    idx_load(3, 3, True)
    pltpu.async_copy(table_hbm.at[ibuf[0]], rows[0], gsem[0])
    pltpu.async_copy(table_hbm.at[ibuf[1]], rows[1], gsem[1])

    def outer(i, carry):
        gbase = i * NBUF
        for b in range(NBUF):
            g = gbase + b
            gather_wait(b)

            @pl.when(g + NBUF < NB)
            def _():
                idx_load(g + NBUF, b, True)

            @pl.when(g >= NBUF)
            def _():
                write_wait(b)

            def row_body(r, c2):
                for j in range(D // L):
                    rows[b][r, pl.ds(j * L, L)] = rows[b][r, pl.ds(j * L, L)] * SCALE
                return c2

            lax.fori_loop(0, BLK, row_body, 0, unroll=4)
            pltpu.async_copy(
                rows[b], out_hbm.at[pl.ds(wid * BLK, BLK), g], wsem[b]
            )

            g2 = g + LOOKAHEAD
            b2 = (b + LOOKAHEAD) % NBUF

            @pl.when(g2 < NB)
            def _():
                idx_wait(b2)
                pltpu.async_copy(table_hbm.at[ibuf[b2]], rows[b2], gsem[b2])

        return carry

    lax.fori_loop(0, NB // NBUF, outer, 0)
    for b in range(NBUF):
        write_wait(b)


def kernel(tokens, table):
    tok_phys = tokens.T.reshape(25, 8, NW, BLK).transpose(0, 2, 1, 3)
    return _emb_lookup(tok_phys, table)
